# baseline (speedup 1.0000x reference)
import functools

import jax
import jax.numpy as jnp
from jax.experimental import pallas as pl
from jax.experimental.pallas import tpu as pltpu

N_EMBD = 128
THRESHOLD = 0.5
ROWS_PER_STEP = 2


def _body(seq_len, x_ref, w_ref, p_ref, bt_ref):
    x = x_ref[...]
    w = w_ref[...]
    R = p_ref.shape[0] // pl.num_programs(0)
    xT = x.T
    kqT = jax.lax.dot_general(
        w, xT, (((0,), (0,)), ((), ())),
        preferred_element_type=jnp.float32,
    )
    kT = kqT[:N_EMBD]
    qT = kqT[N_EMBD:]
    kprevT = pltpu.roll(kT, 1, 1)
    num = jnp.sum(qT * kprevT, axis=0, keepdims=True)
    qq = jnp.sum(qT * qT, axis=0, keepdims=True)
    kk = jnp.sum(kprevT * kprevT, axis=0, keepdims=True)
    cos = num * jax.lax.rsqrt(qq * kk + 1e-30)
    p_row = 0.5 * (1.0 - cos)
    t_idx = jax.lax.broadcasted_iota(jnp.int32, p_row.shape, 1)
    p_row = jnp.where(t_idx % seq_len == 0, 1.0, p_row)
    bt_row = (p_row >= THRESHOLD).astype(jnp.float32)
    i = pl.program_id(0)
    for r in range(R):
        row = i * R + r
        p_ref[pl.ds(row, 1), :] = p_row[:, r * seq_len : (r + 1) * seq_len]
        bt_ref[pl.ds(row, 1), :] = bt_row[:, r * seq_len : (r + 1) * seq_len]


def kernel(x, W):
    Bn, T, C = x.shape
    R = ROWS_PER_STEP
    x2 = x.reshape(Bn * T, C)
    p3, bt3 = pl.pallas_call(
        functools.partial(_body, T),
        grid=(Bn // R,),
        in_specs=[
            pl.BlockSpec((R * T, C), lambda i: (i, 0)),
            pl.BlockSpec((C, 2 * C), lambda i: (0, 0)),
        ],
        out_specs=[
            pl.BlockSpec((Bn, T), lambda i: (0, 0)),
            pl.BlockSpec((Bn, T), lambda i: (0, 0)),
        ],
        out_shape=[
            jax.ShapeDtypeStruct((Bn, T), jnp.float32),
            jax.ShapeDtypeStruct((Bn, T), jnp.float32),
        ],
        compiler_params=pltpu.CompilerParams(
            dimension_semantics=("arbitrary",),
        ),
    )(x2, W)
    return p3, bt3


# speedup vs baseline: 1.0414x; 1.0414x over previous
import functools

import jax
import jax.numpy as jnp
from jax.experimental import pallas as pl
from jax.experimental.pallas import tpu as pltpu

N_EMBD = 128
THRESHOLD = 0.5
ROWS_PER_STEP = 4


def _body(seq_len, x_ref, w_ref, p_ref, bt_ref):
    x = x_ref[...]
    w = w_ref[...]
    R = p_ref.shape[0] // pl.num_programs(0)
    kqT = jax.lax.dot_general(
        w, x, (((0,), (1,)), ((), ())),
        preferred_element_type=jnp.float32,
    )
    kT = kqT[:N_EMBD]
    qT = kqT[N_EMBD:]
    kprevT = pltpu.roll(kT, 1, 1)
    num = jnp.sum(qT * kprevT, axis=0, keepdims=True)
    qq = jnp.sum(qT * qT, axis=0, keepdims=True)
    kk = jnp.sum(kprevT * kprevT, axis=0, keepdims=True)
    cos = num * jax.lax.rsqrt(qq * kk + 1e-30)
    p_row = 0.5 * (1.0 - cos)
    t_idx = jax.lax.broadcasted_iota(jnp.int32, p_row.shape, 1)
    p_row = jnp.where(t_idx % seq_len == 0, 1.0, p_row)
    bt_row = (p_row >= THRESHOLD).astype(jnp.float32)
    i = pl.program_id(0)
    for r in range(R):
        row = i * R + r
        p_ref[pl.ds(row, 1), :] = p_row[:, r * seq_len : (r + 1) * seq_len]
        bt_ref[pl.ds(row, 1), :] = bt_row[:, r * seq_len : (r + 1) * seq_len]


def kernel(x, W):
    Bn, T, C = x.shape
    R = ROWS_PER_STEP
    x2 = x.reshape(Bn * T, C)
    p3, bt3 = pl.pallas_call(
        functools.partial(_body, T),
        grid=(Bn // R,),
        in_specs=[
            pl.BlockSpec((R * T, C), lambda i: (i, 0)),
            pl.BlockSpec((C, 2 * C), lambda i: (0, 0)),
        ],
        out_specs=[
            pl.BlockSpec((Bn, T), lambda i: (0, 0)),
            pl.BlockSpec((Bn, T), lambda i: (0, 0)),
        ],
        out_shape=[
            jax.ShapeDtypeStruct((Bn, T), jnp.float32),
            jax.ShapeDtypeStruct((Bn, T), jnp.float32),
        ],
        compiler_params=pltpu.CompilerParams(
            dimension_semantics=("arbitrary",),
        ),
    )(x2, W)
    return p3, bt3
